# Initial kernel scaffold; baseline (speedup 1.0000x reference)
import jax
import jax.numpy as jnp
from jax.experimental import pallas as pl


def kernel(inputs, codebook):
    raise NotImplementedError("write your pallas kernel here")


# speedup vs baseline: 21.3833x; 21.3833x over previous
import functools

import jax
import jax.numpy as jnp
from jax.experimental import pallas as pl

NQ = 4
K = 1024
D = 256
B = 4096
BLK = 512


def _rvq_kernel(x_ref, cb_ref, nrm_ref, qout_ref, idx_ref):
    r = x_ref[...]
    out = jnp.zeros_like(r)
    kiota = jax.lax.broadcasted_iota(jnp.int32, (BLK, K), 1)
    for i in range(NQ):
        cb = cb_ref[i]
        s = nrm_ref[i][None, :] - 2.0 * jax.lax.dot_general(
            r, cb, (((1,), (1,)), ((), ())),
            precision=jax.lax.Precision.HIGHEST,
            preferred_element_type=jnp.float32)
        idx = jnp.argmin(s, axis=-1).astype(jnp.int32)
        onehot = (kiota == idx[:, None]).astype(jnp.float32)
        q = jax.lax.dot_general(
            onehot, cb, (((1,), (0,)), ((), ())),
            precision=jax.lax.Precision.HIGHEST,
            preferred_element_type=jnp.float32)
        r = r - q
        out = out + q
        idx_ref[:, i] = idx
    qout_ref[...] = out


@jax.jit
def kernel(inputs, codebook):
    nrm = jnp.sum(codebook * codebook, axis=-1)
    grid = (B // BLK,)
    qout, idx = pl.pallas_call(
        _rvq_kernel,
        grid=grid,
        in_specs=[
            pl.BlockSpec((BLK, D), lambda b: (b, 0)),
            pl.BlockSpec((NQ, K, D), lambda b: (0, 0, 0)),
            pl.BlockSpec((NQ, K), lambda b: (0, 0)),
        ],
        out_specs=[
            pl.BlockSpec((BLK, D), lambda b: (b, 0)),
            pl.BlockSpec((BLK, NQ), lambda b: (b, 0)),
        ],
        out_shape=[
            jax.ShapeDtypeStruct((B, D), jnp.float32),
            jax.ShapeDtypeStruct((B, NQ), jnp.int32),
        ],
    )(inputs, codebook, nrm)
    return qout, idx


# speedup vs baseline: 24.5249x; 1.1469x over previous
import jax
import jax.numpy as jnp
from jax.experimental import pallas as pl

NQ = 4
K = 1024
D = 256
B = 4096
BLK = 512


def _ref_order_colsum(sq):
    def half(x):
        acc = x[0:8]
        for j in range(1, 16):
            acc = acc + x[8 * j:8 * j + 8]
        a = acc[0:4] + acc[4:8]
        b = a[0:2] + a[2:4]
        return b[0:1] + b[1:2]
    return half(sq[:128]) + half(sq[128:])


def _rvq_kernel(x_ref, cb_ref, nrm_ref, qout_ref, idx_ref):
    rt = x_ref[...].T
    out_t = jnp.zeros_like(rt)
    kiota = jax.lax.broadcasted_iota(jnp.int32, (K, BLK), 0)
    for i in range(NQ):
        cb = cb_ref[i]
        st = nrm_ref[i][:, None] - 2.0 * jax.lax.dot_general(
            cb, rt, (((1,), (0,)), ((), ())),
            precision=jax.lax.Precision.HIGHEST,
            preferred_element_type=jnp.float32)
        i1 = jnp.argmin(st, axis=0).astype(jnp.int32)[None, :]
        oh1 = kiota == i1
        st2 = jnp.where(oh1, float("inf"), st)
        i2 = jnp.argmin(st2, axis=0).astype(jnp.int32)[None, :]
        oh2 = kiota == i2
        c1 = jax.lax.dot_general(
            cb, oh1.astype(jnp.float32), (((0,), (0,)), ((), ())),
            precision=jax.lax.Precision.HIGHEST,
            preferred_element_type=jnp.float32)
        c2 = jax.lax.dot_general(
            cb, oh2.astype(jnp.float32), (((0,), (0,)), ((), ())),
            precision=jax.lax.Precision.HIGHEST,
            preferred_element_type=jnp.float32)
        e1 = rt - c1
        e2 = rt - c2
        d1 = _ref_order_colsum(e1 * e1)
        d2 = _ref_order_colsum(e2 * e2)
        take2 = (d2 < d1) | ((d2 == d1) & (i2 < i1))
        best = jnp.where(take2, i2, i1)
        q = jnp.where(take2, c2, c1)
        rt = rt - q
        out_t = out_t + q
        idx_ref[i:i + 1, :] = best
    qout_ref[...] = out_t.T


@jax.jit
def kernel(inputs, codebook):
    nrm = jnp.sum(codebook * codebook, axis=-1)
    grid = (B // BLK,)
    qout, idx_t = pl.pallas_call(
        _rvq_kernel,
        grid=grid,
        in_specs=[
            pl.BlockSpec((BLK, D), lambda b: (b, 0)),
            pl.BlockSpec((NQ, K, D), lambda b: (0, 0, 0)),
            pl.BlockSpec((NQ, K), lambda b: (0, 0)),
        ],
        out_specs=[
            pl.BlockSpec((BLK, D), lambda b: (b, 0)),
            pl.BlockSpec((NQ, BLK), lambda b: (0, b)),
        ],
        out_shape=[
            jax.ShapeDtypeStruct((B, D), jnp.float32),
            jax.ShapeDtypeStruct((NQ, B), jnp.int32),
        ],
    )(inputs, codebook, nrm)
    return qout, idx_t.T
